# baseline (speedup 1.0000x reference)
import functools

import jax
import jax.numpy as jnp
from jax import lax
from jax.experimental import pallas as pl
from jax.experimental.pallas import tpu as pltpu
from jax.experimental.pallas import tpu_sc as plsc

_N = 4194304
_NC = 2
_NS = 16
_NW = _NC * _NS
_CHUNK = _N // _NW
_BLK = 16384
_NBLK = _CHUNK // _BLK
_LANES = 16

_mesh = plsc.VectorSubcoreMesh(core_axis_name="c", subcore_axis_name="s")


@functools.partial(
    pl.kernel,
    mesh=_mesh,
    out_type=jax.ShapeDtypeStruct((_N,), jnp.float32),
    scratch_types=[
        pltpu.VMEM((_LANES,), jnp.float32),
        pltpu.VMEM((_BLK,), jnp.int32),
        pltpu.VMEM((_BLK,), jnp.float32),
    ],
)
def _gather_sc(table_hbm, idx_hbm, out_hbm, table_v, idx_v, out_v):
    wid = lax.axis_index("s") * _NC + lax.axis_index("c")
    base = wid * _CHUNK
    pltpu.sync_copy(table_hbm, table_v)
    tab = table_v[...]

    def block_body(b, carry):
        off = base + b * _BLK
        pltpu.sync_copy(idx_hbm.at[pl.ds(off, _BLK)], idx_v)

        def lane_body(i, c):
            s = pl.multiple_of(i * _LANES, _LANES)
            idx = idx_v[pl.ds(s, _LANES)]
            out_v[pl.ds(s, _LANES)] = tab.at[idx].get(
                mode="promise_in_bounds")
            return c

        lax.fori_loop(0, _BLK // _LANES, lane_body, 0, unroll=4)
        pltpu.sync_copy(out_v, out_hbm.at[pl.ds(off, _BLK)])
        return carry

    lax.fori_loop(0, _NBLK, block_body, 0)


def kernel(supervision_weight, index, dummy, bin_num_examples):
    table = jnp.pad(bin_num_examples, (0, _LANES - bin_num_examples.shape[0]))
    return _gather_sc(table, index)


# speedup vs baseline: 2.1201x; 2.1201x over previous
import functools

import jax
import jax.numpy as jnp
from jax import lax
from jax.experimental import pallas as pl
from jax.experimental.pallas import tpu as pltpu
from jax.experimental.pallas import tpu_sc as plsc

_N = 4194304
_NC = 2
_NS = 16
_NW = _NC * _NS
_CHUNK = _N // _NW
_BLK = 16384
_NBLK = _CHUNK // _BLK
_LANES = 16

_mesh = plsc.VectorSubcoreMesh(core_axis_name="c", subcore_axis_name="s")


@functools.partial(
    pl.kernel,
    mesh=_mesh,
    out_type=jax.ShapeDtypeStruct((_N,), jnp.float32),
    scratch_types=[
        pltpu.VMEM((_LANES,), jnp.float32),
        pltpu.VMEM((_BLK,), jnp.int32),
        pltpu.VMEM((_BLK,), jnp.int32),
        pltpu.VMEM((_BLK,), jnp.float32),
        pltpu.VMEM((_BLK,), jnp.float32),
        pltpu.SemaphoreType.DMA,
        pltpu.SemaphoreType.DMA,
        pltpu.SemaphoreType.DMA,
        pltpu.SemaphoreType.DMA,
    ],
)
def _gather_sc(table_hbm, idx_hbm, out_hbm, table_v,
               idx_v0, idx_v1, out_v0, out_v1,
               in_s0, in_s1, out_s0, out_s1):
    wid = lax.axis_index("s") * _NC + lax.axis_index("c")
    base = wid * _CHUNK
    pltpu.sync_copy(table_hbm, table_v)
    tab = table_v[...]

    idx_bufs = (idx_v0, idx_v1)
    out_bufs = (out_v0, out_v1)
    in_sems = (in_s0, in_s1)
    out_sems = (out_s0, out_s1)

    in_copies = {}
    out_copies = {}

    def start_in(b):
        in_copies[b] = pltpu.async_copy(
            idx_hbm.at[pl.ds(base + b * _BLK, _BLK)],
            idx_bufs[b % 2], in_sems[b % 2])

    start_in(0)
    for b in range(_NBLK):
        ib = idx_bufs[b % 2]
        ob = out_bufs[b % 2]
        in_copies.pop(b).wait()
        if b + 1 < _NBLK:
            start_in(b + 1)
        if b >= 2:
            out_copies.pop(b - 2).wait()

        @plsc.parallel_loop(0, _BLK, _LANES, unroll=8)
        def body(s):
            s = pl.multiple_of(s, _LANES)
            idx = ib[pl.ds(s, _LANES)]
            ob[pl.ds(s, _LANES)] = tab.at[idx].get(mode="promise_in_bounds")

        out_copies[b] = pltpu.async_copy(
            ob, out_hbm.at[pl.ds(base + b * _BLK, _BLK)], out_sems[b % 2])

    out_copies.pop(_NBLK - 2).wait()
    out_copies.pop(_NBLK - 1).wait()


def kernel(supervision_weight, index, dummy, bin_num_examples):
    table = jnp.pad(bin_num_examples, (0, _LANES - bin_num_examples.shape[0]))
    return _gather_sc(table, index)
